# baseline (speedup 1.0000x reference)
import math

import jax
import jax.numpy as jnp
from jax.experimental import pallas as pl
from jax.experimental.pallas import tpu as pltpu


def _attractor_body(h_ref, e_ref, w_ref, bias_ref, out_ref, *, seq_len, chunk):
    j = pl.program_id(1)
    hp = jax.lax.dot_general(
        h_ref[0], w_ref[...], (((1,), (1,)), ((), ())),
        preferred_element_type=jnp.float32,
    )
    hp = hp + bias_ref[...]
    ti = j * chunk + jax.lax.broadcasted_iota(jnp.int32, (chunk, 1), 0)
    t = ti.astype(jnp.float32)
    decay = jnp.exp((seq_len - 1.0 - t) * (-math.pi / seq_len))
    hpw = hp * decay
    contrib = jax.lax.dot_general(
        hpw, e_ref[0], (((0,), (0,)), ((), ())),
        preferred_element_type=jnp.float32,
    )

    @pl.when(j == 0)
    def _init():
        out_ref[0] = contrib

    @pl.when(j != 0)
    def _accum():
        out_ref[0] += contrib


def kernel(hidden_states, positional_encodings, W, b):
    bsz, seq_len, d_model = hidden_states.shape
    d_state = W.shape[0]
    chunk = 2048
    assert seq_len % chunk == 0
    bias2d = b.reshape(1, d_state)

    import functools
    body = functools.partial(_attractor_body, seq_len=seq_len, chunk=chunk)

    return pl.pallas_call(
        body,
        out_shape=jax.ShapeDtypeStruct((bsz, d_state, d_model), jnp.float32),
        grid=(bsz, seq_len // chunk),
        in_specs=[
            pl.BlockSpec((1, chunk, d_model), lambda i, j: (i, j, 0)),
            pl.BlockSpec((1, chunk, d_model), lambda i, j: (i, j, 0)),
            pl.BlockSpec((d_state, d_model), lambda i, j: (0, 0)),
            pl.BlockSpec((1, d_state), lambda i, j: (0, 0)),
        ],
        out_specs=pl.BlockSpec((1, d_state, d_model), lambda i, j: (i, 0, 0)),
        compiler_params=pltpu.CompilerParams(
            dimension_semantics=("parallel", "arbitrary"),
        ),
        name="attractor_state",
    )(hidden_states, positional_encodings, W, bias2d)


# speedup vs baseline: 1.1383x; 1.1383x over previous
import functools
import math

import jax
import jax.numpy as jnp
from jax.experimental import pallas as pl
from jax.experimental.pallas import tpu as pltpu


def _attractor_body(h_ref, e_ref, w_ref, bias_ref, out_ref, m_acc, s_acc,
                    *, seq_len, chunk):
    j = pl.program_id(1)
    nj = pl.num_programs(1)
    ti = j * chunk + jax.lax.broadcasted_iota(jnp.int32, (chunk, 1), 0)
    decay = jnp.exp((seq_len - 1.0 - ti.astype(jnp.float32))
                    * (-math.pi / seq_len))
    hw = h_ref[0] * decay
    e = e_ref[0]
    contrib = jax.lax.dot_general(
        hw, e, (((0,), (0,)), ((), ())),
        preferred_element_type=jnp.float32,
    )
    s_contrib = jnp.sum(decay * e, axis=0, keepdims=True)

    @pl.when(j == 0)
    def _init():
        m_acc[...] = contrib
        s_acc[...] = s_contrib

    @pl.when(j != 0)
    def _accum():
        m_acc[...] += contrib
        s_acc[...] += s_contrib

    @pl.when(j == nj - 1)
    def _finish():
        out_ref[0] = jax.lax.dot_general(
            w_ref[...], m_acc[...], (((1,), (0,)), ((), ())),
            preferred_element_type=jnp.float32,
        ) + bias_ref[...] * s_acc[...]


def kernel(hidden_states, positional_encodings, W, b):
    bsz, seq_len, d_model = hidden_states.shape
    d_state = W.shape[0]
    chunk = 2048
    assert seq_len % chunk == 0
    bias_col = b.reshape(d_state, 1)

    body = functools.partial(_attractor_body, seq_len=seq_len, chunk=chunk)

    return pl.pallas_call(
        body,
        out_shape=jax.ShapeDtypeStruct((bsz, d_state, d_model), jnp.float32),
        grid=(bsz, seq_len // chunk),
        in_specs=[
            pl.BlockSpec((1, chunk, d_model), lambda i, j: (i, j, 0)),
            pl.BlockSpec((1, chunk, d_model), lambda i, j: (i, j, 0)),
            pl.BlockSpec((d_state, d_model), lambda i, j: (0, 0)),
            pl.BlockSpec((d_state, 1), lambda i, j: (0, 0)),
        ],
        out_specs=pl.BlockSpec((1, d_state, d_model), lambda i, j: (i, 0, 0)),
        scratch_shapes=[
            pltpu.VMEM((d_model, d_model), jnp.float32),
            pltpu.VMEM((1, d_model), jnp.float32),
        ],
        compiler_params=pltpu.CompilerParams(
            dimension_semantics=("parallel", "arbitrary"),
        ),
        name="attractor_state",
    )(hidden_states, positional_encodings, W, bias_col)


# speedup vs baseline: 1.1640x; 1.0226x over previous
import functools
import math

import jax
import jax.numpy as jnp
from jax.experimental import pallas as pl
from jax.experimental.pallas import tpu as pltpu


def _attractor_body(h_ref, e_ref, w_ref, bias_ref, out_ref, m_acc, s_acc,
                    *, seq_len, chunk):
    j = pl.program_id(1)
    nj = pl.num_programs(1)
    ti = j * chunk + jax.lax.broadcasted_iota(jnp.int32, (chunk, 1), 0)
    decay = jnp.exp((seq_len - 1.0 - ti.astype(jnp.float32))
                    * (-math.pi / seq_len))
    hw = h_ref[0] * decay
    e = e_ref[0]
    contrib = jax.lax.dot_general(
        hw, e, (((0,), (0,)), ((), ())),
        preferred_element_type=jnp.float32,
    )
    s_contrib = jnp.sum(decay * e, axis=0, keepdims=True)

    @pl.when(j == 0)
    def _init():
        m_acc[...] = contrib
        s_acc[...] = s_contrib

    @pl.when(j != 0)
    def _accum():
        m_acc[...] += contrib
        s_acc[...] += s_contrib

    @pl.when(j == nj - 1)
    def _finish():
        out_ref[0] = jax.lax.dot_general(
            w_ref[...], m_acc[...], (((1,), (0,)), ((), ())),
            preferred_element_type=jnp.float32,
        ) + bias_ref[...] * s_acc[...]


def kernel(hidden_states, positional_encodings, W, b):
    bsz, seq_len, d_model = hidden_states.shape
    d_state = W.shape[0]
    chunk = 4096
    assert seq_len % chunk == 0
    bias_col = b.reshape(d_state, 1)

    body = functools.partial(_attractor_body, seq_len=seq_len, chunk=chunk)

    return pl.pallas_call(
        body,
        out_shape=jax.ShapeDtypeStruct((bsz, d_state, d_model), jnp.float32),
        grid=(bsz, seq_len // chunk),
        in_specs=[
            pl.BlockSpec((1, chunk, d_model), lambda i, j: (i, j, 0)),
            pl.BlockSpec((1, chunk, d_model), lambda i, j: (i, j, 0)),
            pl.BlockSpec((d_state, d_model), lambda i, j: (0, 0)),
            pl.BlockSpec((d_state, 1), lambda i, j: (0, 0)),
        ],
        out_specs=pl.BlockSpec((1, d_state, d_model), lambda i, j: (i, 0, 0)),
        scratch_shapes=[
            pltpu.VMEM((d_model, d_model), jnp.float32),
            pltpu.VMEM((1, d_model), jnp.float32),
        ],
        compiler_params=pltpu.CompilerParams(
            dimension_semantics=("parallel", "arbitrary"),
        ),
        name="attractor_state",
    )(hidden_states, positional_encodings, W, bias_col)
